# baseline (speedup 1.0000x reference)
import jax
import jax.numpy as jnp
from jax import lax
from jax.experimental import pallas as pl
from jax.experimental.pallas import tpu as pltpu
from jax.experimental.pallas import tpu_sc as plsc

N = 10000
D = 128
NCLS = 40
L = 16
NC = 2
NS = 16
NW = NC * NS
K = 32
SB = 8
NSUP = 41
NBLK = NSUP * SB
NBLK_ALLOC = NBLK + SB
CHUNK = NBLK_ALLOC * K
NPAD = 10112
RPT = NPAD // NS
MBLK = 1000


def _tc1_body(x_ref, w_ref, av_ref, h_ref, aa_ref):
    h = jnp.dot(x_ref[...], w_ref[...], preferred_element_type=jnp.float32)
    h_ref[...] = h
    aa_ref[...] = jnp.dot(h, av_ref[...], preferred_element_type=jnp.float32)


def _tc1(x, w, av):
    return pl.pallas_call(
        _tc1_body,
        grid=(N // MBLK,),
        in_specs=[
            pl.BlockSpec((MBLK, D), lambda i: (i, 0)),
            pl.BlockSpec((D, D), lambda i: (0, 0)),
            pl.BlockSpec((D, 8), lambda i: (0, 0)),
        ],
        out_specs=[
            pl.BlockSpec((MBLK, D), lambda i: (i, 0)),
            pl.BlockSpec((MBLK, 8), lambda i: (i, 0)),
        ],
        out_shape=[
            jax.ShapeDtypeStruct((N, D), jnp.float32),
            jax.ShapeDtypeStruct((N, 8), jnp.float32),
        ],
    )(x, w, av)


def _merge(acc_ref, den_ref, b_ref):
    s = acc_ref[0] + acc_ref[1]
    den = jnp.sum(den_ref[...], axis=1, keepdims=True)
    return jnp.maximum(s / den + b_ref[...], 0.0)


def _tc2_body(acc_ref, den_ref, b_ref, w_ref, av_ref, h_ref, aa_ref):
    x2 = _merge(acc_ref, den_ref, b_ref)
    h = jnp.dot(x2, w_ref[...], preferred_element_type=jnp.float32)
    h_ref[...] = h
    aa_ref[...] = jnp.dot(h, av_ref[...], preferred_element_type=jnp.float32)


def _tc2(acc, den, b, w, av):
    return pl.pallas_call(
        _tc2_body,
        grid=(N // MBLK,),
        in_specs=[
            pl.BlockSpec((NC, MBLK, D), lambda i: (0, i, 0)),
            pl.BlockSpec((MBLK, NW), lambda i: (i, 0)),
            pl.BlockSpec((1, D), lambda i: (0, 0)),
            pl.BlockSpec((D, D), lambda i: (0, 0)),
            pl.BlockSpec((D, 8), lambda i: (0, 0)),
        ],
        out_specs=[
            pl.BlockSpec((MBLK, D), lambda i: (i, 0)),
            pl.BlockSpec((MBLK, 8), lambda i: (i, 0)),
        ],
        out_shape=[
            jax.ShapeDtypeStruct((N, D), jnp.float32),
            jax.ShapeDtypeStruct((N, 8), jnp.float32),
        ],
    )(acc, den, b, w, av)


def _tc3_body(acc_ref, den_ref, b_ref, wc_ref, bc_ref, out_ref, sacc):
    i = pl.program_id(0)

    @pl.when(i == 0)
    def _():
        sacc[...] = jnp.zeros_like(sacc)

    x3 = _merge(acc_ref, den_ref, b_ref)
    sacc[...] += jnp.sum(x3, axis=0, keepdims=True)

    @pl.when(i == pl.num_programs(0) - 1)
    def _():
        m = sacc[...] * (1.0 / N)
        out_ref[...] = (
            jnp.dot(m, wc_ref[...], preferred_element_type=jnp.float32)
            + bc_ref[...]
        )


def _tc3(acc, den, b, wc, bc):
    return pl.pallas_call(
        _tc3_body,
        grid=(N // MBLK,),
        in_specs=[
            pl.BlockSpec((NC, MBLK, D), lambda i: (0, i, 0)),
            pl.BlockSpec((MBLK, NW), lambda i: (i, 0)),
            pl.BlockSpec((1, D), lambda i: (0, 0)),
            pl.BlockSpec((D, NCLS), lambda i: (0, 0)),
            pl.BlockSpec((1, NCLS), lambda i: (0, 0)),
        ],
        out_specs=pl.BlockSpec((1, NCLS), lambda i: (0, 0)),
        out_shape=jax.ShapeDtypeStruct((1, NCLS), jnp.float32),
        scratch_shapes=[pltpu.VMEM((1, D), jnp.float32)],
    )(acc, den, b, wc, bc)


def _sc_body(src_hbm, dst_hbm, as_hbm, ad_hbm, h_hbm,
             acc_hbm, den_hbm,
             sb0, sb1, db0, db1,
             r0, r1, r2, r3,
             asv, adv, denv, ebuf, acc_sp,
             *sems):
    cid = lax.axis_index("c")
    sid = lax.axis_index("s")
    wid = sid * NC + cid
    sbig = (sb0, sb1)
    dbig = (db0, db1)
    rows = (r0, r1, r2, r3)
    isem = sems[0:2]
    gsem = sems[2:6]
    ssem = sems[6:10]
    z16 = jnp.zeros((16,), jnp.float32)
    s_hbm = src_hbm.at[wid].at[0]
    d_hbm = dst_hbm.at[wid].at[0]

    pltpu.sync_copy(as_hbm, asv)
    pltpu.sync_copy(ad_hbm, adv)

    def _zden(i, c):
        denv[0, pl.ds(i * 16, 16)] = z16
        return c

    lax.fori_loop(0, NPAD // 16, _zden, 0)

    def _zrow(r, c):
        rv = rows[0].at[r]
        for k in range(D // 16):
            rv[pl.ds(k * 16, 16)] = z16
        return c

    lax.fori_loop(0, K, _zrow, 0)
    base = sid * RPT
    nfull = RPT // K
    rem = RPT - nfull * K
    for q in range(nfull):
        pltpu.sync_copy(rows[0], acc_sp.at[pl.ds(base + q * K, K)])
    if rem:
        pltpu.sync_copy(rows[0].at[pl.ds(0, rem)],
                        acc_sp.at[pl.ds(base + nfull * K, rem)])

    def start_idx(S, t):
        pltpu.async_copy(s_hbm.at[S], sbig[t], isem[t])
        pltpu.async_copy(d_hbm.at[S], dbig[t], isem[t])

    def wait_idx(S, t):
        pltpu.make_async_copy(s_hbm.at[S], sbig[t], isem[t]).wait()
        pltpu.make_async_copy(d_hbm.at[S], dbig[t], isem[t]).wait()

    def gref(j, t):
        return sbig[t].at[0].at[pl.ds(j * K, K)]

    def start_gather(j, t, s):
        pltpu.async_copy(h_hbm.at[gref(j, t)], rows[s], gsem[s])

    def wait_gather(j, t, s):
        pltpu.make_async_copy(h_hbm.at[gref(j, t)], rows[s],
                              gsem[s]).wait()

    def start_scatter(j, t, s):
        pltpu.async_copy(rows[s], acc_sp.at[dbig[t].at[j]], ssem[s],
                         add=True)

    def wait_scatter(j, t, s):
        pltpu.make_async_copy(rows[s], acc_sp.at[dbig[t].at[j]],
                              ssem[s]).wait()

    zi16 = jnp.zeros((16,), jnp.int32)

    def compute(j, t, s):
        rview = rows[s]
        sview = sbig[t]
        dview = dbig[t]

        for g16 in range(K // 16):
            sv = sview[0, pl.ds(j * K + g16 * 16, 16)]
            dv = dview[j, pl.ds(g16 * 16, 16)]
            a = plsc.load_gather(asv, [sv]) + plsc.load_gather(adv, [dv])
            a = jnp.where(a >= 0.0, a, a * 0.2)
            e = jnp.exp(a)
            plsc.addupdate_scatter(denv, [zi16, dv], e)
            ebuf[pl.ds(g16 * 16, 16)] = e

        zsplat = lax.shift_right_arithmetic(sview[0, pl.ds(0, 16)], 31)

        @plsc.parallel_loop(0, K, 1, unroll=4)
        def _scale(r):
            ev = plsc.load_gather(ebuf, [zsplat + r])
            rrow = rview.at[r]
            for k in range(D // 16):
                rrow[pl.ds(k * 16, 16)] = rrow[pl.ds(k * 16, 16)] * ev

    def super_iter(S, t, t1, first_super):
        for j in range(SB):
            s = j % 4
            wait_gather(j, t, s)
            if j == 2:
                start_idx(S + 1, t1)
            if j == SB - 1:
                wait_idx(S + 1, t1)
                start_gather(0, t1, (j + 1) % 4)
            else:
                start_gather(j + 1, t, (j + 1) % 4)
            compute(j, t, s)
            start_scatter(j, t, s)
            if not (first_super and j < 2):
                wait_scatter((j + 2) % 4, t, (j + 2) % 4)

    start_idx(0, 0)
    wait_idx(0, 0)
    start_gather(0, 0, 0)
    plsc.subcore_barrier()
    super_iter(0, 0, 1, True)

    def outer(p, c):
        super_iter(1 + 2 * p, 1, 0, False)
        super_iter(2 + 2 * p, 0, 1, False)
        return c

    lax.fori_loop(0, (NSUP - 1) // 2, outer, 0)

    wait_gather(0, 1, 0)
    wait_scatter(SB - 2, 0, 2)
    wait_scatter(SB - 1, 0, 3)
    plsc.subcore_barrier()

    pltpu.sync_copy(acc_sp.at[pl.ds(base, RPT)],
                    acc_hbm.at[cid].at[pl.ds(base, RPT)])
    pltpu.sync_copy(denv, den_hbm.at[wid])


_sc_edge = pl.kernel(
    _sc_body,
    out_type=(
        jax.ShapeDtypeStruct((NC, NPAD, D), jnp.float32),
        jax.ShapeDtypeStruct((NW, 1, NPAD), jnp.float32),
    ),
    mesh=plsc.VectorSubcoreMesh(
        core_axis_name="c", subcore_axis_name="s",
        num_cores=NC, num_subcores=NS),
    compiler_params=pltpu.CompilerParams(needs_layout_passes=False),
    scratch_types=(
        (pltpu.VMEM((1, SB * K), jnp.int32),) * 2
        + (pltpu.VMEM((SB, K), jnp.int32),) * 2
        + (pltpu.VMEM((K, D), jnp.float32),) * 4
        + (
            pltpu.VMEM((NPAD,), jnp.float32),
            pltpu.VMEM((NPAD,), jnp.float32),
            pltpu.VMEM((1, NPAD), jnp.float32),
            pltpu.VMEM((K,), jnp.float32),
            pltpu.VMEM_SHARED((NPAD, D), jnp.float32),
        )
        + (pltpu.SemaphoreType.DMA,) * 10
    ),
)


def kernel(x, edge_index, W1, a_src1, a_dst1, b1, W2, a_src2, a_dst2, b2,
           Wc, bc):
    idx = edge_index.astype(jnp.int32)
    loop = jnp.arange(N, dtype=jnp.int32)
    src = jnp.concatenate([idx[0], loop])
    dst = jnp.concatenate([idx[1], loop])
    e_tot = src.shape[0]
    e_used = NW * NBLK * K
    assert e_used >= e_tot
    src = jnp.pad(src, (0, e_used - e_tot)).reshape(NW, NBLK * K)
    dst = jnp.pad(dst, (0, e_used - e_tot),
                  constant_values=N).reshape(NW, NBLK * K)
    src_t = jnp.concatenate(
        [src, jnp.zeros((NW, SB * K), jnp.int32)], axis=1).reshape(
            NW, 1, NSUP + 1, 1, SB * K)
    dst_t = jnp.concatenate(
        [dst, jnp.full((NW, SB * K), N, jnp.int32)], axis=1).reshape(
            NW, 1, NSUP + 1, SB, K)

    def pack_av(a_s, a_d):
        return jnp.stack(
            [a_s, a_d] + [jnp.zeros((D,), jnp.float32)] * 6, axis=1)

    def pads(aa):
        return (jnp.pad(aa[:, 0], (0, NPAD - N)),
                jnp.pad(aa[:, 1], (0, NPAD - N)))

    h1, aa1 = _tc1(x, W1, pack_av(a_src1, a_dst1))
    as1, ad1 = pads(aa1)
    acc1, den1 = _sc_edge(src_t, dst_t, as1, ad1, h1)
    h2, aa2 = _tc2(acc1, den1.reshape(NW, NPAD)[:, :N].T, b1.reshape(1, D),
                   W2, pack_av(a_src2, a_dst2))
    as2, ad2 = pads(aa2)
    acc2, den2 = _sc_edge(src_t, dst_t, as2, ad2, h2)
    return _tc3(acc2, den2.reshape(NW, NPAD)[:, :N].T, b2.reshape(1, D),
                Wc, bc.reshape(1, NCLS))


# speedup vs baseline: 1.0104x; 1.0104x over previous
import jax
import jax.numpy as jnp
from jax import lax
from jax.experimental import pallas as pl
from jax.experimental.pallas import tpu as pltpu
from jax.experimental.pallas import tpu_sc as plsc

N = 10000
D = 128
NCLS = 40
L = 16
NC = 2
NS = 16
NW = NC * NS
K = 32
SB = 8
NSUP = 41
NBLK = NSUP * SB
NBLK_ALLOC = NBLK + SB
CHUNK = NBLK_ALLOC * K
NPAD = 10112
RPT = NPAD // NS
MBLK = 1000


def _tc1_body(x_ref, w_ref, av_ref, h_ref, aa_ref):
    h = jnp.dot(x_ref[...], w_ref[...], preferred_element_type=jnp.float32)
    h_ref[...] = h
    aa_ref[...] = jnp.dot(h, av_ref[...], preferred_element_type=jnp.float32)


def _tc1(x, w, av):
    return pl.pallas_call(
        _tc1_body,
        grid=(N // MBLK,),
        in_specs=[
            pl.BlockSpec((MBLK, D), lambda i: (i, 0)),
            pl.BlockSpec((D, D), lambda i: (0, 0)),
            pl.BlockSpec((D, 8), lambda i: (0, 0)),
        ],
        out_specs=[
            pl.BlockSpec((MBLK, D), lambda i: (i, 0)),
            pl.BlockSpec((MBLK, 8), lambda i: (i, 0)),
        ],
        out_shape=[
            jax.ShapeDtypeStruct((N, D), jnp.float32),
            jax.ShapeDtypeStruct((N, 8), jnp.float32),
        ],
    )(x, w, av)


def _merge(acc_ref, den_ref, b_ref):
    s = acc_ref[0] + acc_ref[1]
    den = jnp.sum(den_ref[...], axis=1, keepdims=True)
    return jnp.maximum(s / den + b_ref[...], 0.0)


def _tc2_body(acc_ref, den_ref, b_ref, w_ref, av_ref, h_ref, aa_ref):
    x2 = _merge(acc_ref, den_ref, b_ref)
    h = jnp.dot(x2, w_ref[...], preferred_element_type=jnp.float32)
    h_ref[...] = h
    aa_ref[...] = jnp.dot(h, av_ref[...], preferred_element_type=jnp.float32)


def _tc2(acc, den, b, w, av):
    return pl.pallas_call(
        _tc2_body,
        grid=(N // MBLK,),
        in_specs=[
            pl.BlockSpec((NC, MBLK, D), lambda i: (0, i, 0)),
            pl.BlockSpec((MBLK, NW), lambda i: (i, 0)),
            pl.BlockSpec((1, D), lambda i: (0, 0)),
            pl.BlockSpec((D, D), lambda i: (0, 0)),
            pl.BlockSpec((D, 8), lambda i: (0, 0)),
        ],
        out_specs=[
            pl.BlockSpec((MBLK, D), lambda i: (i, 0)),
            pl.BlockSpec((MBLK, 8), lambda i: (i, 0)),
        ],
        out_shape=[
            jax.ShapeDtypeStruct((N, D), jnp.float32),
            jax.ShapeDtypeStruct((N, 8), jnp.float32),
        ],
    )(acc, den, b, w, av)


def _tc3_body(acc_ref, den_ref, b_ref, wc_ref, bc_ref, out_ref, sacc):
    i = pl.program_id(0)

    @pl.when(i == 0)
    def _():
        sacc[...] = jnp.zeros_like(sacc)

    x3 = _merge(acc_ref, den_ref, b_ref)
    sacc[...] += jnp.sum(x3, axis=0, keepdims=True)

    @pl.when(i == pl.num_programs(0) - 1)
    def _():
        m = sacc[...] * (1.0 / N)
        out_ref[...] = (
            jnp.dot(m, wc_ref[...], preferred_element_type=jnp.float32)
            + bc_ref[...]
        )


def _tc3(acc, den, b, wc, bc):
    return pl.pallas_call(
        _tc3_body,
        grid=(N // MBLK,),
        in_specs=[
            pl.BlockSpec((NC, MBLK, D), lambda i: (0, i, 0)),
            pl.BlockSpec((MBLK, NW), lambda i: (i, 0)),
            pl.BlockSpec((1, D), lambda i: (0, 0)),
            pl.BlockSpec((D, NCLS), lambda i: (0, 0)),
            pl.BlockSpec((1, NCLS), lambda i: (0, 0)),
        ],
        out_specs=pl.BlockSpec((1, NCLS), lambda i: (0, 0)),
        out_shape=jax.ShapeDtypeStruct((1, NCLS), jnp.float32),
        scratch_shapes=[pltpu.VMEM((1, D), jnp.float32)],
    )(acc, den, b, wc, bc)


def _sc_body(src_hbm, dst_hbm, as_hbm, ad_hbm, h_hbm,
             acc_hbm, den_hbm,
             sb0, sb1, db0, db1,
             r0, r1, r2, r3,
             asv, adv, denv, ebuf, acc_sp,
             *sems):
    cid = lax.axis_index("c")
    sid = lax.axis_index("s")
    wid = sid * NC + cid
    sbig = (sb0, sb1)
    dbig = (db0, db1)
    rows = (r0, r1, r2, r3)
    isem = sems[0:2]
    gsem = sems[2:6]
    ssem = sems[6:10]
    gsem2 = sems[10:14]
    z16 = jnp.zeros((16,), jnp.float32)
    s_hbm = src_hbm.at[wid].at[0]
    d_hbm = dst_hbm.at[wid].at[0]

    pltpu.sync_copy(as_hbm, asv)
    pltpu.sync_copy(ad_hbm, adv)

    def _zden(i, c):
        denv[0, pl.ds(i * 16, 16)] = z16
        return c

    lax.fori_loop(0, NPAD // 16, _zden, 0)

    def _zrow(r, c):
        rv = rows[0].at[r]
        for k in range(D // 16):
            rv[pl.ds(k * 16, 16)] = z16
        return c

    lax.fori_loop(0, K, _zrow, 0)
    base = sid * RPT
    nfull = RPT // K
    rem = RPT - nfull * K
    for q in range(nfull):
        pltpu.sync_copy(rows[0], acc_sp.at[pl.ds(base + q * K, K)])
    if rem:
        pltpu.sync_copy(rows[0].at[pl.ds(0, rem)],
                        acc_sp.at[pl.ds(base + nfull * K, rem)])

    def start_idx(S, t):
        pltpu.async_copy(s_hbm.at[S], sbig[t], isem[t])
        pltpu.async_copy(d_hbm.at[S], dbig[t], isem[t])

    def wait_idx(S, t):
        pltpu.make_async_copy(s_hbm.at[S], sbig[t], isem[t]).wait()
        pltpu.make_async_copy(d_hbm.at[S], dbig[t], isem[t]).wait()

    H = K // 2

    def gref(j, t, h):
        return sbig[t].at[0].at[pl.ds(j * K + h * H, H)]

    def start_gather(j, t, s):
        pltpu.async_copy(h_hbm.at[gref(j, t, 0)],
                         rows[s].at[pl.ds(0, H)], gsem[s])
        pltpu.async_copy(h_hbm.at[gref(j, t, 1)],
                         rows[s].at[pl.ds(H, H)], gsem2[s])

    def wait_gather(j, t, s):
        pltpu.make_async_copy(h_hbm.at[gref(j, t, 0)],
                              rows[s].at[pl.ds(0, H)], gsem[s]).wait()
        pltpu.make_async_copy(h_hbm.at[gref(j, t, 1)],
                              rows[s].at[pl.ds(H, H)], gsem2[s]).wait()

    def start_scatter(j, t, s):
        pltpu.async_copy(rows[s], acc_sp.at[dbig[t].at[j]], ssem[s],
                         add=True)

    def wait_scatter(j, t, s):
        pltpu.make_async_copy(rows[s], acc_sp.at[dbig[t].at[j]],
                              ssem[s]).wait()

    zi16 = jnp.zeros((16,), jnp.int32)

    def compute(j, t, s):
        rview = rows[s]
        sview = sbig[t]
        dview = dbig[t]

        for g16 in range(K // 16):
            sv = sview[0, pl.ds(j * K + g16 * 16, 16)]
            dv = dview[j, pl.ds(g16 * 16, 16)]
            a = plsc.load_gather(asv, [sv]) + plsc.load_gather(adv, [dv])
            a = jnp.where(a >= 0.0, a, a * 0.2)
            e = jnp.exp(a)
            plsc.addupdate_scatter(denv, [zi16, dv], e)
            ebuf[pl.ds(g16 * 16, 16)] = e

        zsplat = lax.shift_right_arithmetic(sview[0, pl.ds(0, 16)], 31)

        @plsc.parallel_loop(0, K, 1, unroll=4)
        def _scale(r):
            ev = plsc.load_gather(ebuf, [zsplat + r])
            rrow = rview.at[r]
            for k in range(D // 16):
                rrow[pl.ds(k * 16, 16)] = rrow[pl.ds(k * 16, 16)] * ev

    def super_iter(S, t, t1, first_super):
        for j in range(SB):
            s = j % 4
            wait_gather(j, t, s)
            if j == 2:
                start_idx(S + 1, t1)
            if j == SB - 1:
                wait_idx(S + 1, t1)
                start_gather(0, t1, (j + 1) % 4)
            else:
                start_gather(j + 1, t, (j + 1) % 4)
            compute(j, t, s)
            start_scatter(j, t, s)
            if not (first_super and j < 2):
                wait_scatter((j + 2) % 4, t, (j + 2) % 4)

    start_idx(0, 0)
    wait_idx(0, 0)
    start_gather(0, 0, 0)
    plsc.subcore_barrier()
    super_iter(0, 0, 1, True)

    def outer(p, c):
        super_iter(1 + 2 * p, 1, 0, False)
        super_iter(2 + 2 * p, 0, 1, False)
        return c

    lax.fori_loop(0, (NSUP - 1) // 2, outer, 0)

    wait_gather(0, 1, 0)
    wait_scatter(SB - 2, 0, 2)
    wait_scatter(SB - 1, 0, 3)
    plsc.subcore_barrier()

    pltpu.sync_copy(acc_sp.at[pl.ds(base, RPT)],
                    acc_hbm.at[cid].at[pl.ds(base, RPT)])
    pltpu.sync_copy(denv, den_hbm.at[wid])


_sc_edge = pl.kernel(
    _sc_body,
    out_type=(
        jax.ShapeDtypeStruct((NC, NPAD, D), jnp.float32),
        jax.ShapeDtypeStruct((NW, 1, NPAD), jnp.float32),
    ),
    mesh=plsc.VectorSubcoreMesh(
        core_axis_name="c", subcore_axis_name="s",
        num_cores=NC, num_subcores=NS),
    compiler_params=pltpu.CompilerParams(needs_layout_passes=False),
    scratch_types=(
        (pltpu.VMEM((1, SB * K), jnp.int32),) * 2
        + (pltpu.VMEM((SB, K), jnp.int32),) * 2
        + (pltpu.VMEM((K, D), jnp.float32),) * 4
        + (
            pltpu.VMEM((NPAD,), jnp.float32),
            pltpu.VMEM((NPAD,), jnp.float32),
            pltpu.VMEM((1, NPAD), jnp.float32),
            pltpu.VMEM((K,), jnp.float32),
            pltpu.VMEM_SHARED((NPAD, D), jnp.float32),
        )
        + (pltpu.SemaphoreType.DMA,) * 14
    ),
)


def kernel(x, edge_index, W1, a_src1, a_dst1, b1, W2, a_src2, a_dst2, b2,
           Wc, bc):
    idx = edge_index.astype(jnp.int32)
    loop = jnp.arange(N, dtype=jnp.int32)
    src = jnp.concatenate([idx[0], loop])
    dst = jnp.concatenate([idx[1], loop])
    e_tot = src.shape[0]
    e_used = NW * NBLK * K
    assert e_used >= e_tot
    src = jnp.pad(src, (0, e_used - e_tot)).reshape(NW, NBLK * K)
    dst = jnp.pad(dst, (0, e_used - e_tot),
                  constant_values=N).reshape(NW, NBLK * K)
    src_t = jnp.concatenate(
        [src, jnp.zeros((NW, SB * K), jnp.int32)], axis=1).reshape(
            NW, 1, NSUP + 1, 1, SB * K)
    dst_t = jnp.concatenate(
        [dst, jnp.full((NW, SB * K), N, jnp.int32)], axis=1).reshape(
            NW, 1, NSUP + 1, SB, K)

    def pack_av(a_s, a_d):
        return jnp.stack(
            [a_s, a_d] + [jnp.zeros((D,), jnp.float32)] * 6, axis=1)

    def pads(aa):
        return (jnp.pad(aa[:, 0], (0, NPAD - N)),
                jnp.pad(aa[:, 1], (0, NPAD - N)))

    h1, aa1 = _tc1(x, W1, pack_av(a_src1, a_dst1))
    as1, ad1 = pads(aa1)
    acc1, den1 = _sc_edge(src_t, dst_t, as1, ad1, h1)
    h2, aa2 = _tc2(acc1, den1.reshape(NW, NPAD)[:, :N].T, b1.reshape(1, D),
                   W2, pack_av(a_src2, a_dst2))
    as2, ad2 = pads(aa2)
    acc2, den2 = _sc_edge(src_t, dst_t, as2, ad2, h2)
    return _tc3(acc2, den2.reshape(NW, NPAD)[:, :N].T, b2.reshape(1, D),
                Wc, bc.reshape(1, NCLS))


# speedup vs baseline: 1.1129x; 1.1014x over previous
import jax
import jax.numpy as jnp
from jax import lax
from jax.experimental import pallas as pl
from jax.experimental.pallas import tpu as pltpu
from jax.experimental.pallas import tpu_sc as plsc

N = 10000
D = 128
NCLS = 40
L = 16
NC = 2
NS = 16
NW = NC * NS
K = 32
NBLK = 326
NBLK_ALLOC = NBLK + 2
CHUNK = NBLK_ALLOC * K
NPAD = 10112
RPT = NPAD // NS
MBLK = 1000


def _tc1_body(x_ref, w_ref, av_ref, h_ref, aa_ref):
    h = jnp.dot(x_ref[...], w_ref[...], preferred_element_type=jnp.float32)
    h_ref[...] = h
    aa_ref[...] = jnp.dot(h, av_ref[...], preferred_element_type=jnp.float32)


def _tc1(x, w, av):
    return pl.pallas_call(
        _tc1_body,
        grid=(N // MBLK,),
        in_specs=[
            pl.BlockSpec((MBLK, D), lambda i: (i, 0)),
            pl.BlockSpec((D, D), lambda i: (0, 0)),
            pl.BlockSpec((D, 8), lambda i: (0, 0)),
        ],
        out_specs=[
            pl.BlockSpec((MBLK, D), lambda i: (i, 0)),
            pl.BlockSpec((MBLK, 8), lambda i: (i, 0)),
        ],
        out_shape=[
            jax.ShapeDtypeStruct((N, D), jnp.float32),
            jax.ShapeDtypeStruct((N, 8), jnp.float32),
        ],
    )(x, w, av)


def _merge(acc_ref, den_ref, b_ref):
    s = acc_ref[0] + acc_ref[1]
    den = jnp.sum(den_ref[...], axis=1, keepdims=True)
    return jnp.maximum(s / den + b_ref[...], 0.0)


def _tc2_body(acc_ref, den_ref, b_ref, w_ref, av_ref, h_ref, aa_ref):
    x2 = _merge(acc_ref, den_ref, b_ref)
    h = jnp.dot(x2, w_ref[...], preferred_element_type=jnp.float32)
    h_ref[...] = h
    aa_ref[...] = jnp.dot(h, av_ref[...], preferred_element_type=jnp.float32)


def _tc2(acc, den, b, w, av):
    return pl.pallas_call(
        _tc2_body,
        grid=(N // MBLK,),
        in_specs=[
            pl.BlockSpec((NC, MBLK, D), lambda i: (0, i, 0)),
            pl.BlockSpec((MBLK, NW), lambda i: (i, 0)),
            pl.BlockSpec((1, D), lambda i: (0, 0)),
            pl.BlockSpec((D, D), lambda i: (0, 0)),
            pl.BlockSpec((D, 8), lambda i: (0, 0)),
        ],
        out_specs=[
            pl.BlockSpec((MBLK, D), lambda i: (i, 0)),
            pl.BlockSpec((MBLK, 8), lambda i: (i, 0)),
        ],
        out_shape=[
            jax.ShapeDtypeStruct((N, D), jnp.float32),
            jax.ShapeDtypeStruct((N, 8), jnp.float32),
        ],
    )(acc, den, b, w, av)


def _tc3_body(acc_ref, den_ref, b_ref, wc_ref, bc_ref, out_ref, sacc):
    i = pl.program_id(0)

    @pl.when(i == 0)
    def _():
        sacc[...] = jnp.zeros_like(sacc)

    x3 = _merge(acc_ref, den_ref, b_ref)
    sacc[...] += jnp.sum(x3, axis=0, keepdims=True)

    @pl.when(i == pl.num_programs(0) - 1)
    def _():
        m = sacc[...] * (1.0 / N)
        out_ref[...] = (
            jnp.dot(m, wc_ref[...], preferred_element_type=jnp.float32)
            + bc_ref[...]
        )


def _tc3(acc, den, b, wc, bc):
    return pl.pallas_call(
        _tc3_body,
        grid=(N // MBLK,),
        in_specs=[
            pl.BlockSpec((NC, MBLK, D), lambda i: (0, i, 0)),
            pl.BlockSpec((MBLK, NW), lambda i: (i, 0)),
            pl.BlockSpec((1, D), lambda i: (0, 0)),
            pl.BlockSpec((D, NCLS), lambda i: (0, 0)),
            pl.BlockSpec((1, NCLS), lambda i: (0, 0)),
        ],
        out_specs=pl.BlockSpec((1, NCLS), lambda i: (0, 0)),
        out_shape=jax.ShapeDtypeStruct((1, NCLS), jnp.float32),
        scratch_shapes=[pltpu.VMEM((1, D), jnp.float32)],
    )(acc, den, b, wc, bc)


def _sc_body(src_hbm, dst_hbm, as_hbm, ad_hbm, h_hbm,
             acc_hbm, den_hbm,
             si0, si1, si2, si3, di0, di1, di2, di3,
             r0, r1, r2, r3,
             asv, adv, denv, ebuf, acc_sp,
             *sems):
    cid = lax.axis_index("c")
    sid = lax.axis_index("s")
    wid = sid * NC + cid
    sidx = (si0, si1, si2, si3)
    didx = (di0, di1, di2, di3)
    rows = (r0, r1, r2, r3)
    isem = sems[0:4]
    gsem = sems[4:8]
    ssem = sems[8:12]
    z16 = jnp.zeros((16,), jnp.float32)
    srow_hbm = src_hbm.at[wid].at[0]
    drow_hbm = dst_hbm.at[wid].at[0]

    pltpu.sync_copy(as_hbm, asv)
    pltpu.sync_copy(ad_hbm, adv)

    def _zden(i, c):
        denv[0, pl.ds(i * 16, 16)] = z16
        return c

    lax.fori_loop(0, NPAD // 16, _zden, 0)

    def _zrow(r, c):
        rv = rows[0].at[r]
        for k in range(D // 16):
            rv[pl.ds(k * 16, 16)] = z16
        return c

    lax.fori_loop(0, K, _zrow, 0)
    base = sid * RPT
    nfull = RPT // K
    rem = RPT - nfull * K
    for q in range(nfull):
        pltpu.sync_copy(rows[0], acc_sp.at[pl.ds(base + q * K, K)])
    if rem:
        pltpu.sync_copy(rows[0].at[pl.ds(0, rem)],
                        acc_sp.at[pl.ds(base + nfull * K, rem)])

    def start_idx(b, s):
        pltpu.async_copy(srow_hbm.at[pl.ds(b * K, K)], sidx[s], isem[s])
        pltpu.async_copy(drow_hbm.at[pl.ds(b * K, K)], didx[s], isem[s])

    def wait_idx(b, s):
        pltpu.make_async_copy(srow_hbm.at[pl.ds(b * K, K)], sidx[s],
                              isem[s]).wait()
        pltpu.make_async_copy(drow_hbm.at[pl.ds(b * K, K)], didx[s],
                              isem[s]).wait()

    def start_gather(s):
        pltpu.async_copy(h_hbm.at[sidx[s]], rows[s], gsem[s])

    def wait_gather(s):
        pltpu.make_async_copy(h_hbm.at[sidx[s]], rows[s], gsem[s]).wait()

    def start_scatter(s):
        pltpu.async_copy(rows[s], acc_sp.at[didx[s]], ssem[s], add=True)

    def wait_scatter(s):
        pltpu.make_async_copy(rows[s], acc_sp.at[didx[s]], ssem[s]).wait()

    zi16 = jnp.zeros((16,), jnp.int32)

    def compute(s):
        rview = rows[s]
        sview = sidx[s]
        dview = didx[s]

        for g16 in range(K // 16):
            sv = sview[pl.ds(g16 * 16, 16)]
            dv = dview[pl.ds(g16 * 16, 16)]
            a = plsc.load_gather(asv, [sv]) + plsc.load_gather(adv, [dv])
            a = jnp.where(a >= 0.0, a, a * 0.2)
            e = jnp.exp(a)
            plsc.addupdate_scatter(denv, [zi16, dv], e)
            ebuf[pl.ds(g16 * 16, 16)] = e

        zsplat = lax.shift_right_arithmetic(sview[pl.ds(0, 16)], 31)

        @plsc.parallel_loop(0, K, 1, unroll=4)
        def _scale(r):
            ev = plsc.load_gather(ebuf, [zsplat + r])
            rrow = rview.at[r]
            for k in range(D // 16):
                rrow[pl.ds(k * 16, 16)] = rrow[pl.ds(k * 16, 16)] * ev

    def iteration(b, s, first):
        wait_gather(s)
        s1 = (s + 1) % 4
        s2 = (s + 2) % 4
        wait_idx(b + 1, s1)
        start_gather(s1)
        compute(s)
        start_scatter(s)
        if not first:
            wait_scatter(s2)
        start_idx(b + 2, s2)

    start_idx(0, 0)
    start_idx(1, 1)
    wait_idx(0, 0)
    start_gather(0)
    plsc.subcore_barrier()
    iteration(0, 0, True)
    iteration(1, 1, True)

    def outer(i, c):
        b = 2 + i * 4
        iteration(b + 0, 2, False)
        iteration(b + 1, 3, False)
        iteration(b + 2, 0, False)
        iteration(b + 3, 1, False)
        return c

    lax.fori_loop(0, (NBLK - 2) // 4, outer, 0)

    wait_gather(NBLK % 4)
    wait_idx(NBLK + 1, (NBLK + 1) % 4)
    wait_scatter((NBLK - 2) % 4)
    wait_scatter((NBLK - 1) % 4)
    plsc.subcore_barrier()

    pltpu.sync_copy(acc_sp.at[pl.ds(base, RPT)],
                    acc_hbm.at[cid].at[pl.ds(base, RPT)])
    pltpu.sync_copy(denv, den_hbm.at[wid])


_sc_edge = pl.kernel(
    _sc_body,
    out_type=(
        jax.ShapeDtypeStruct((NC, NPAD, D), jnp.float32),
        jax.ShapeDtypeStruct((NW, 1, NPAD), jnp.float32),
    ),
    mesh=plsc.VectorSubcoreMesh(
        core_axis_name="c", subcore_axis_name="s",
        num_cores=NC, num_subcores=NS),
    compiler_params=pltpu.CompilerParams(needs_layout_passes=False),
    scratch_types=(
        (pltpu.VMEM((K,), jnp.int32),) * 4
        + (pltpu.VMEM((K,), jnp.int32),) * 4
        + (pltpu.VMEM((K, D), jnp.float32),) * 4
        + (
            pltpu.VMEM((NPAD,), jnp.float32),
            pltpu.VMEM((NPAD,), jnp.float32),
            pltpu.VMEM((1, NPAD), jnp.float32),
            pltpu.VMEM((K,), jnp.float32),
            pltpu.VMEM_SHARED((NPAD, D), jnp.float32),
        )
        + (pltpu.SemaphoreType.DMA,) * 12
    ),
)


def kernel(x, edge_index, W1, a_src1, a_dst1, b1, W2, a_src2, a_dst2, b2,
           Wc, bc):
    idx = edge_index.astype(jnp.int32)
    loop = jnp.arange(N, dtype=jnp.int32)
    src = jnp.concatenate([idx[0], loop])
    dst = jnp.concatenate([idx[1], loop])
    e_tot = src.shape[0]
    e_used = NW * NBLK * K
    assert e_used >= e_tot
    src = jnp.pad(src, (0, e_used - e_tot)).reshape(NW, NBLK * K)
    dst = jnp.pad(dst, (0, e_used - e_tot),
                  constant_values=N).reshape(NW, NBLK * K)
    src_t = jnp.concatenate(
        [src, jnp.zeros((NW, 2 * K), jnp.int32)], axis=1).reshape(
            NW, 1, CHUNK)
    dst_t = jnp.concatenate(
        [dst, jnp.full((NW, 2 * K), N, jnp.int32)], axis=1).reshape(
            NW, 1, CHUNK)

    def pack_av(a_s, a_d):
        return jnp.stack(
            [a_s, a_d] + [jnp.zeros((D,), jnp.float32)] * 6, axis=1)

    def pads(aa):
        return (jnp.pad(aa[:, 0], (0, NPAD - N)),
                jnp.pad(aa[:, 1], (0, NPAD - N)))

    h1, aa1 = _tc1(x, W1, pack_av(a_src1, a_dst1))
    as1, ad1 = pads(aa1)
    acc1, den1 = _sc_edge(src_t, dst_t, as1, ad1, h1)
    h2, aa2 = _tc2(acc1, den1.reshape(NW, NPAD)[:, :N].T, b1.reshape(1, D),
                   W2, pack_av(a_src2, a_dst2))
    as2, ad2 = pads(aa2)
    acc2, den2 = _sc_edge(src_t, dst_t, as2, ad2, h2)
    return _tc3(acc2, den2.reshape(NW, NPAD)[:, :N].T, b2.reshape(1, D),
                Wc, bc.reshape(1, NCLS))
